# baseline (speedup 1.0000x reference)
import functools
import math

import jax
import jax.numpy as jnp
from jax import lax
from jax.experimental import pallas as pl
from jax.experimental.pallas import tpu as pltpu
from jax.experimental.pallas import tpu_sc as plsc

EMBED = 32
SCALE = math.sqrt(float(EMBED))
LANES = 16


def _detile_call(V, NC, NS):
    NW = NC * NS
    ET = EMBED // 8
    W = 4096
    VB = V // W
    NTASK = ET * VB
    V_TAIL0 = VB * W
    TAIL_FULL = ((V - V_TAIL0) // 128) * 128
    TAIL_REM = V - V_TAIL0 - TAIL_FULL
    TV = (V + 127) // 128
    NB = 2

    mesh = plsc.VectorSubcoreMesh(core_axis_name="c", subcore_axis_name="s")

    @functools.partial(
        pl.kernel,
        mesh=mesh,
        out_type=jax.ShapeDtypeStruct((ET * TV * 8, 128), jnp.float32),
        compiler_params=pltpu.CompilerParams(use_tc_tiling_on_sc=True),
        scratch_types=[pltpu.VMEM((NB, 8, W), jnp.float32)]
        + [pltpu.SemaphoreType.DMA] * (2 * NB),
    )
    def k1a(tt_hbm, inter_hbm, stage_v, *sems):
        g_sems, w_sems = sems[:NB], sems[NB:]
        wid = lax.axis_index("s") * NC + lax.axis_index("c")
        n_my = (NTASK - 1 - wid) // NW + 1

        def read_dma(j, b):
            bi = wid + j * NW
            et = bi % ET
            v0 = (bi // ET) * W
            v0 = pl.multiple_of(v0, W)
            return pltpu.async_copy(
                tt_hbm.at[pl.ds(et * 8, 8), pl.ds(v0, W)],
                stage_v.at[b],
                g_sems[b],
            )

        def write_tiles(j, u):
            bi = wid + j * NW
            et = bi % ET
            vt0 = (bi // ET) * (W // 128)
            for c in range(W // 128):
                line = (et * TV + vt0 + c) * 8
                line = pl.multiple_of(line, 8)
                _ = pltpu.async_copy(
                    stage_v.at[u, pl.ds(0, 8), pl.ds(c * 128, 128)],
                    inter_hbm.at[pl.ds(line, 8)],
                    w_sems[u],
                )

        def drain_writes(u):
            for c in range(W // 128):
                pltpu.make_async_copy(
                    stage_v.at[u, pl.ds(0, 8), pl.ds(0, 128)],
                    inter_hbm.at[pl.ds(0, 8)],
                    w_sems[u],
                ).wait()

        def body(j2, carry):
            for u in range(NB):
                j = j2 * NB + u

                @pl.when(j + 1 < n_my)
                def _(j=j, b=(u + 1) % NB):
                    @pl.when(j + 1 >= NB)
                    def _():
                        drain_writes(b)

                    _ = read_dma(j + 1, b)

                @pl.when(j < n_my)
                def _(j=j, u=u):
                    pltpu.make_async_copy(
                        tt_hbm.at[pl.ds(0, 8), pl.ds(0, W)],
                        stage_v.at[u],
                        g_sems[u],
                    ).wait()
                    write_tiles(j, u)
            return carry

        @pl.when(n_my > 0)
        def _():
            _ = read_dma(0, 0)

        lax.fori_loop(0, (n_my + NB - 1) // NB, body, 0)

        for u in range(NB):
            @pl.when(n_my >= u + 1)
            def _(u=u):
                drain_writes(u)


        if TAIL_FULL:
            @pl.when(wid < ET)
            def _():
                et = wid
                if TAIL_FULL:
                    nt = TAIL_FULL // 128
                    pltpu.sync_copy(
                        tt_hbm.at[pl.ds(et * 8, 8), pl.ds(V_TAIL0, TAIL_FULL)],
                        stage_v.at[0, pl.ds(0, 8), pl.ds(0, TAIL_FULL)],
                    )
                    for c in range(nt):
                        pltpu.sync_copy(
                            stage_v.at[0, pl.ds(0, 8), pl.ds(c * 128, 128)],
                            inter_hbm.at[
                                pl.ds((et * TV + V_TAIL0 // 128 + c) * 8, 8)
                            ],
                        )

    return k1a


def _scale_table_call(V, NC, NS):
    NW = NC * NS
    VT = V // 128
    V_REM = V - VT * 128
    TV = (V + 127) // 128
    ET = EMBED // 8
    NB = 4
    LA = 2

    mesh = plsc.VectorSubcoreMesh(core_axis_name="c", subcore_axis_name="s")

    @functools.partial(
        pl.kernel,
        mesh=mesh,
        out_type=jax.ShapeDtypeStruct((V * EMBED // 128, 128), jnp.float32),
        compiler_params=pltpu.CompilerParams(
            use_tc_tiling_on_sc=False, needs_layout_passes=False
        ),
        scratch_types=[
            pltpu.VMEM((NB, EMBED, 128), jnp.float32),
            pltpu.VMEM((NB, EMBED, 128), jnp.float32),
            pltpu.VMEM((128, EMBED), jnp.float32),
        ]
        + [pltpu.SemaphoreType.DMA] * (2 * NB),
    )
    def k1b(inter_hbm, tail_hbm, scratch_hbm, stage_v, o_v, tail_v, *sems):
        g_sems, w_sems = sems[:NB], sems[NB:]
        wid = lax.axis_index("s") * NC + lax.axis_index("c")
        n_my = (VT - 1 - wid) // NW + 1
        iota = lax.iota(jnp.int32, LANES)
        e_lo = iota
        e_hi = iota + LANES

        def read_dma(j, b):
            vt = wid + j * NW
            for et in range(ET):
                line = (et * TV + vt) * 8
                line = pl.multiple_of(line, 8)
                _ = pltpu.async_copy(
                    inter_hbm.at[pl.ds(line, 8)],
                    stage_v.at[b, pl.ds(et * 8, 8)],
                    g_sems[b],
                )

        def wait_read(b):
            for et in range(ET):
                pltpu.make_async_copy(
                    inter_hbm.at[pl.ds(0, 8)],
                    stage_v.at[b, pl.ds(0, 8)],
                    g_sems[b],
                ).wait()

        def transform(b, n_vl):
            bvec = jnp.full((LANES,), b, jnp.int32)
            for f0 in range(0, n_vl * EMBED, LANES):
                vl = f0 // EMBED
                e_vec = e_lo if (f0 % EMBED) == 0 else e_hi
                vals = plsc.load_gather(
                    stage_v, [bvec, e_vec, jnp.full((LANES,), vl, jnp.int32)]
                )
                o_v[b, f0 // 128, pl.ds(f0 % 128, LANES)] = vals * SCALE

        def body(j2, carry):
            for u in range(NB):
                j = j2 * NB + u

                @pl.when(j + LA < n_my)
                def _(j=j, b=(u + LA) % NB):
                    read_dma(j + LA, b)

                @pl.when(jnp.logical_and(j >= NB, j < n_my))
                def _(u=u):
                    pltpu.make_async_copy(
                        o_v.at[u], scratch_hbm.at[pl.ds(0, EMBED)], w_sems[u]
                    ).wait()

                @pl.when(j < n_my)
                def _(j=j, u=u):
                    wait_read(u)
                    transform(u, 128)
                    line0 = (wid + j * NW) * EMBED
                    line0 = pl.multiple_of(line0, EMBED)
                    _ = pltpu.async_copy(
                        o_v.at[u], scratch_hbm.at[pl.ds(line0, EMBED)],
                        w_sems[u],
                    )
            return carry

        for j in range(LA):
            @pl.when(j < n_my)
            def _(j=j):
                read_dma(j, j % NB)

        lax.fori_loop(0, (n_my + NB - 1) // NB, body, 0)

        for u in range(NB):
            @pl.when(n_my >= u + 1)
            def _(u=u):
                pltpu.make_async_copy(
                    o_v.at[u], scratch_hbm.at[pl.ds(0, EMBED)], w_sems[u]
                ).wait()

        if V_REM:
            @pl.when(wid == 0)
            def _():
                pltpu.sync_copy(tail_hbm, tail_v.at[pl.ds(0, V_REM)])
                for r in range(V_REM):
                    for c0 in range(0, EMBED, LANES):
                        f = r * EMBED + c0
                        o_v[0, f // 128, pl.ds(f % 128, LANES)] = (
                            tail_v[r, pl.ds(c0, LANES)] * SCALE
                        )
                pltpu.sync_copy(
                    o_v.at[0, pl.ds(0, V_REM * EMBED // 128), pl.ds(0, 128)],
                    scratch_hbm.at[pl.ds(VT * EMBED, V_REM * EMBED // 128)],
                )

    return k1b


def _gather_call(S, T, V, NC, NS):
    NW = NC * NS
    SB = S // 128
    assert SB == NW
    ET = EMBED // 8
    NB = 4
    LA = 3

    mesh = plsc.VectorSubcoreMesh(core_axis_name="c", subcore_axis_name="s")

    @functools.partial(
        pl.kernel,
        mesh=mesh,
        out_type=jax.ShapeDtypeStruct((T, ET, SB, 8, 128), jnp.float32),
        compiler_params=pltpu.CompilerParams(
            use_tc_tiling_on_sc=False, needs_layout_passes=False
        ),
        scratch_types=[
            pltpu.VMEM((128, T), jnp.int32),
            pltpu.VMEM((T, 128), jnp.int32),
            pltpu.VMEM((NB, 128, EMBED), jnp.float32),
            pltpu.VMEM((NB, ET, 8, 128), jnp.float32),
        ]
        + [pltpu.SemaphoreType.DMA] * (2 * NB),
    )
    def k2(tok_hbm, table_hbm, out_hbm, stage_v, tpose_v, rows_v, b_v, *sems):
        g_sems, w_sems = sems[:NB], sems[NB:]
        wid = lax.axis_index("s") * NC + lax.axis_index("c")
        iota = lax.iota(jnp.int32, LANES)

        pltpu.sync_copy(tok_hbm.at[pl.ds(wid * 128, 128)], stage_v)
        s_vecs = [iota + c0 for c0 in range(0, 128, LANES)]

        def tpose_body(t, carry):
            tvec = jnp.full((LANES,), t, jnp.int32)
            for ci, sv in enumerate(s_vecs):
                vals = plsc.load_gather(stage_v, [sv, tvec])
                tpose_v[t, pl.ds(ci * LANES, LANES)] = vals
            return carry

        lax.fori_loop(0, T, tpose_body, 0, unroll=2)

        def gather(t, br):
            return pltpu.async_copy(
                table_hbm.at[tpose_v.at[t]], rows_v.at[br], g_sems[br]
            )

        def transform(br):
            bvec = jnp.full((LANES,), br, jnp.int32)
            for et in range(ET):
                for ei in range(8):
                    evec = jnp.full((LANES,), et * 8 + ei, jnp.int32)
                    for ci, sv in enumerate(s_vecs):
                        vals = plsc.load_gather(rows_v, [bvec, sv, evec])
                        b_v[br, et, ei, pl.ds(ci * LANES, LANES)] = vals

        def body(t2, carry):
            for u in range(NB):
                t = t2 * NB + u

                @pl.when(t + LA < T)
                def _(t=t, br=(u + LA) % NB):
                    _ = gather(t + LA, br)

                @pl.when(t >= NB)
                def _(u=u):
                    for et in range(ET):
                        pltpu.make_async_copy(
                            b_v.at[u, et], out_hbm.at[0, et, wid], w_sems[u]
                        ).wait()

                pltpu.make_async_copy(
                    table_hbm.at[tpose_v.at[0]], rows_v.at[u], g_sems[u]
                ).wait()
                transform(u)
                for et in range(ET):
                    _ = pltpu.async_copy(
                        b_v.at[u, et], out_hbm.at[t, et, wid], w_sems[u]
                    )
            return carry

        for t in range(LA):
            _ = gather(t, t)
        lax.fori_loop(0, T // NB, body, 0)
        for u in range(NB):
            for et in range(ET):
                pltpu.make_async_copy(
                    b_v.at[u, et], out_hbm.at[0, et, wid], w_sems[u]
                ).wait()

    return k2


def kernel(tokens, table):
    S, T = tokens.shape
    V = table.shape[0]
    if tokens.dtype != jnp.int32:
        tokens = tokens.astype(jnp.int32)
    info = plsc.get_sparse_core_info()
    NC, NS = info.num_cores, info.num_subcores

    inter = _detile_call(V, NC, NS)(table.T)
    v_full = (V // 128) * 128
    tail = table[v_full:]
    scratch = _scale_table_call(V, NC, NS)(inter, tail)
    scaled = scratch.reshape(V, EMBED)
    out5 = _gather_call(S, T, V, NC, NS)(tokens, scaled)
    return out5.transpose(2, 4, 0, 1, 3).reshape(S, T, EMBED)


# speedup vs baseline: 1.2493x; 1.2493x over previous
import functools
import math

import jax
import jax.numpy as jnp
from jax import lax
from jax.experimental import pallas as pl
from jax.experimental.pallas import tpu as pltpu
from jax.experimental.pallas import tpu_sc as plsc

EMBED = 32
SCALE = math.sqrt(float(EMBED))
LANES = 16


def _detile_call(V, NC, NS):
    NW = NC * NS
    ET = EMBED // 8
    W = 4096
    VB = V // W
    NTASK = ET * VB
    V_TAIL0 = VB * W
    TAIL_FULL = ((V - V_TAIL0) // 128) * 128
    TAIL_REM = V - V_TAIL0 - TAIL_FULL
    TV = (V + 127) // 128
    NB = 2

    mesh = plsc.VectorSubcoreMesh(core_axis_name="c", subcore_axis_name="s")

    @functools.partial(
        pl.kernel,
        mesh=mesh,
        out_type=jax.ShapeDtypeStruct((ET * TV * 8, 128), jnp.float32),
        compiler_params=pltpu.CompilerParams(use_tc_tiling_on_sc=True),
        scratch_types=[pltpu.VMEM((NB, 8, W), jnp.float32)]
        + [pltpu.SemaphoreType.DMA] * (2 * NB),
    )
    def k1a(tt_hbm, inter_hbm, stage_v, *sems):
        g_sems, w_sems = sems[:NB], sems[NB:]
        wid = lax.axis_index("s") * NC + lax.axis_index("c")
        n_my = (NTASK - 1 - wid) // NW + 1

        def read_dma(j, b):
            bi = wid + j * NW
            et = bi % ET
            v0 = (bi // ET) * W
            v0 = pl.multiple_of(v0, W)
            return pltpu.async_copy(
                tt_hbm.at[pl.ds(et * 8, 8), pl.ds(v0, W)],
                stage_v.at[b],
                g_sems[b],
            )

        def write_tiles(j, u):
            bi = wid + j * NW
            et = bi % ET
            vt0 = (bi // ET) * (W // 128)
            for c in range(W // 128):
                line = (et * TV + vt0 + c) * 8
                line = pl.multiple_of(line, 8)
                _ = pltpu.async_copy(
                    stage_v.at[u, pl.ds(0, 8), pl.ds(c * 128, 128)],
                    inter_hbm.at[pl.ds(line, 8)],
                    w_sems[u],
                )

        def drain_writes(u):
            for c in range(W // 128):
                pltpu.make_async_copy(
                    stage_v.at[u, pl.ds(0, 8), pl.ds(0, 128)],
                    inter_hbm.at[pl.ds(0, 8)],
                    w_sems[u],
                ).wait()

        def body(j2, carry):
            for u in range(NB):
                j = j2 * NB + u

                @pl.when(j + 1 < n_my)
                def _(j=j, b=(u + 1) % NB):
                    @pl.when(j + 1 >= NB)
                    def _():
                        drain_writes(b)

                    _ = read_dma(j + 1, b)

                @pl.when(j < n_my)
                def _(j=j, u=u):
                    pltpu.make_async_copy(
                        tt_hbm.at[pl.ds(0, 8), pl.ds(0, W)],
                        stage_v.at[u],
                        g_sems[u],
                    ).wait()
                    write_tiles(j, u)
            return carry

        @pl.when(n_my > 0)
        def _():
            _ = read_dma(0, 0)

        lax.fori_loop(0, (n_my + NB - 1) // NB, body, 0)

        for u in range(NB):
            @pl.when(n_my >= u + 1)
            def _(u=u):
                drain_writes(u)


        if TAIL_FULL:
            @pl.when(wid < ET)
            def _():
                et = wid
                if TAIL_FULL:
                    nt = TAIL_FULL // 128
                    pltpu.sync_copy(
                        tt_hbm.at[pl.ds(et * 8, 8), pl.ds(V_TAIL0, TAIL_FULL)],
                        stage_v.at[0, pl.ds(0, 8), pl.ds(0, TAIL_FULL)],
                    )
                    for c in range(nt):
                        pltpu.sync_copy(
                            stage_v.at[0, pl.ds(0, 8), pl.ds(c * 128, 128)],
                            inter_hbm.at[
                                pl.ds((et * TV + V_TAIL0 // 128 + c) * 8, 8)
                            ],
                        )

    return k1a


def _scale_table_call(V, NC, NS):
    NW = NC * NS
    VT = V // 128
    V_REM = V - VT * 128
    TV = (V + 127) // 128
    ET = EMBED // 8
    NB = 4
    LA = 2

    mesh = plsc.VectorSubcoreMesh(core_axis_name="c", subcore_axis_name="s")

    @functools.partial(
        pl.kernel,
        mesh=mesh,
        out_type=jax.ShapeDtypeStruct((V * EMBED // 128, 128), jnp.float32),
        compiler_params=pltpu.CompilerParams(
            use_tc_tiling_on_sc=False, needs_layout_passes=False
        ),
        scratch_types=[
            pltpu.VMEM((NB, EMBED, 128), jnp.float32),
            pltpu.VMEM((NB, EMBED, 128), jnp.float32),
            pltpu.VMEM((128, EMBED), jnp.float32),
            pltpu.VMEM((EMBED, 133), jnp.float32),
        ]
        + [pltpu.SemaphoreType.DMA] * (2 * NB),
    )
    def k1b(inter_hbm, tail_hbm, scratch_hbm, stage_v, o_v, tail_v, sp_v,
            *sems):
        g_sems, w_sems = sems[:NB], sems[NB:]
        wid = lax.axis_index("s") * NC + lax.axis_index("c")
        n_my = (VT - 1 - wid) // NW + 1
        iota = lax.iota(jnp.int32, LANES)
        e_lo = iota
        e_hi = iota + LANES

        def read_dma(j, b):
            vt = wid + j * NW
            for et in range(ET):
                line = (et * TV + vt) * 8
                line = pl.multiple_of(line, 8)
                _ = pltpu.async_copy(
                    inter_hbm.at[pl.ds(line, 8)],
                    stage_v.at[b, pl.ds(et * 8, 8)],
                    g_sems[b],
                )

        def wait_read(b):
            for et in range(ET):
                pltpu.make_async_copy(
                    inter_hbm.at[pl.ds(0, 8)],
                    stage_v.at[b, pl.ds(0, 8)],
                    g_sems[b],
                ).wait()

        def transform(b, n_vl):
            for e in range(EMBED):
                for c0 in range(0, n_vl, LANES):
                    sp_v[e, pl.ds(c0, LANES)] = stage_v[b, e, pl.ds(c0, LANES)]
            for f0 in range(0, n_vl * EMBED, LANES):
                vl = f0 // EMBED
                e_vec = e_lo if (f0 % EMBED) == 0 else e_hi
                vals = plsc.load_gather(
                    sp_v, [e_vec, jnp.full((LANES,), vl, jnp.int32)]
                )
                o_v[b, f0 // 128, pl.ds(f0 % 128, LANES)] = vals * SCALE

        def body(j2, carry):
            for u in range(NB):
                j = j2 * NB + u

                @pl.when(j + LA < n_my)
                def _(j=j, b=(u + LA) % NB):
                    read_dma(j + LA, b)

                @pl.when(jnp.logical_and(j >= NB, j < n_my))
                def _(u=u):
                    pltpu.make_async_copy(
                        o_v.at[u], scratch_hbm.at[pl.ds(0, EMBED)], w_sems[u]
                    ).wait()

                @pl.when(j < n_my)
                def _(j=j, u=u):
                    wait_read(u)
                    transform(u, 128)
                    line0 = (wid + j * NW) * EMBED
                    line0 = pl.multiple_of(line0, EMBED)
                    _ = pltpu.async_copy(
                        o_v.at[u], scratch_hbm.at[pl.ds(line0, EMBED)],
                        w_sems[u],
                    )
            return carry

        for j in range(LA):
            @pl.when(j < n_my)
            def _(j=j):
                read_dma(j, j % NB)

        lax.fori_loop(0, (n_my + NB - 1) // NB, body, 0)

        for u in range(NB):
            @pl.when(n_my >= u + 1)
            def _(u=u):
                pltpu.make_async_copy(
                    o_v.at[u], scratch_hbm.at[pl.ds(0, EMBED)], w_sems[u]
                ).wait()

        if V_REM:
            @pl.when(wid == 0)
            def _():
                pltpu.sync_copy(tail_hbm, tail_v.at[pl.ds(0, V_REM)])
                for r in range(V_REM):
                    for c0 in range(0, EMBED, LANES):
                        f = r * EMBED + c0
                        o_v[0, f // 128, pl.ds(f % 128, LANES)] = (
                            tail_v[r, pl.ds(c0, LANES)] * SCALE
                        )
                pltpu.sync_copy(
                    o_v.at[0, pl.ds(0, V_REM * EMBED // 128), pl.ds(0, 128)],
                    scratch_hbm.at[pl.ds(VT * EMBED, V_REM * EMBED // 128)],
                )

    return k1b


def _gather_call(S, T, V, NC, NS):
    NW = NC * NS
    SB = S // 128
    assert SB == NW
    ET = EMBED // 8
    NB = 4
    LA = 3

    mesh = plsc.VectorSubcoreMesh(core_axis_name="c", subcore_axis_name="s")

    TP = T + 1
    scratch_types_list = [
            pltpu.VMEM((128, TP), jnp.int32),
            pltpu.VMEM((T, 128), jnp.int32),
            pltpu.VMEM((NB, 128, EMBED), jnp.float32),
            pltpu.VMEM((128, EMBED + 1), jnp.float32),
            pltpu.VMEM((NB, ET, 8, 128), jnp.float32),
    ]

    @functools.partial(
        pl.kernel,
        mesh=mesh,
        out_type=jax.ShapeDtypeStruct((T, ET, SB, 8, 128), jnp.float32),
        compiler_params=pltpu.CompilerParams(
            use_tc_tiling_on_sc=False, needs_layout_passes=False
        ),
        scratch_types=scratch_types_list
        + [pltpu.SemaphoreType.DMA] * (2 * NB),
    )
    def k2(tok_hbm, table_hbm, out_hbm, stage_v, tpose_v, rows_v, rp_v, b_v,
           *sems):
        g_sems, w_sems = sems[:NB], sems[NB:]
        wid = lax.axis_index("s") * NC + lax.axis_index("c")
        iota = lax.iota(jnp.int32, LANES)

        pltpu.sync_copy(
            tok_hbm.at[pl.ds(wid * 128, 128)],
            stage_v.at[pl.ds(0, 128), pl.ds(0, T)],
        )
        s_vecs = [iota + c0 for c0 in range(0, 128, LANES)]

        def tpose_body(t, carry):
            tvec = jnp.full((LANES,), t, jnp.int32)
            for ci, sv in enumerate(s_vecs):
                vals = plsc.load_gather(stage_v, [sv, tvec])
                tpose_v[t, pl.ds(ci * LANES, LANES)] = vals
            return carry

        lax.fori_loop(0, T, tpose_body, 0, unroll=2)

        def gather(t, br):
            return pltpu.async_copy(
                table_hbm.at[tpose_v.at[t]], rows_v.at[br], g_sems[br]
            )

        def transform(br):
            for si in range(0, 128):
                for c0 in range(0, EMBED, LANES):
                    rp_v[si, pl.ds(c0, LANES)] = rows_v[
                        br, si, pl.ds(c0, LANES)
                    ]
            for et in range(ET):
                for ei in range(8):
                    evec = jnp.full((LANES,), et * 8 + ei, jnp.int32)
                    for ci, sv in enumerate(s_vecs):
                        vals = plsc.load_gather(rp_v, [sv, evec])
                        b_v[br, et, ei, pl.ds(ci * LANES, LANES)] = vals

        def body(t2, carry):
            for u in range(NB):
                t = t2 * NB + u

                @pl.when(t + LA < T)
                def _(t=t, br=(u + LA) % NB):
                    _ = gather(t + LA, br)

                @pl.when(t >= NB)
                def _(u=u):
                    for et in range(ET):
                        pltpu.make_async_copy(
                            b_v.at[u, et], out_hbm.at[0, et, wid], w_sems[u]
                        ).wait()

                pltpu.make_async_copy(
                    table_hbm.at[tpose_v.at[0]], rows_v.at[u], g_sems[u]
                ).wait()
                transform(u)
                for et in range(ET):
                    _ = pltpu.async_copy(
                        b_v.at[u, et], out_hbm.at[t, et, wid], w_sems[u]
                    )
            return carry

        for t in range(LA):
            _ = gather(t, t)
        lax.fori_loop(0, T // NB, body, 0)
        for u in range(NB):
            for et in range(ET):
                pltpu.make_async_copy(
                    b_v.at[u, et], out_hbm.at[0, et, wid], w_sems[u]
                ).wait()

    return k2


def kernel(tokens, table):
    S, T = tokens.shape
    V = table.shape[0]
    if tokens.dtype != jnp.int32:
        tokens = tokens.astype(jnp.int32)
    info = plsc.get_sparse_core_info()
    NC, NS = info.num_cores, info.num_subcores

    inter = _detile_call(V, NC, NS)(table.T)
    v_full = (V // 128) * 128
    tail = table[v_full:]
    scratch = _scale_table_call(V, NC, NS)(inter, tail)
    scaled = scratch.reshape(V, EMBED)
    out5 = _gather_call(S, T, V, NC, NS)(tokens, scaled)
    return out5.transpose(2, 4, 0, 1, 3).reshape(S, T, EMBED)


# speedup vs baseline: 1.3460x; 1.0774x over previous
import functools
import math

import jax
import jax.numpy as jnp
from jax import lax
from jax.experimental import pallas as pl
from jax.experimental.pallas import tpu as pltpu
from jax.experimental.pallas import tpu_sc as plsc

EMBED = 32
SCALE = math.sqrt(float(EMBED))
LANES = 16


def _detile_call(V, NC, NS):
    NW = NC * NS
    ET = EMBED // 8
    W = 4096
    VB = V // W
    NTASK = ET * VB
    V_TAIL0 = VB * W
    TAIL_FULL = ((V - V_TAIL0) // 128) * 128
    TAIL_REM = V - V_TAIL0 - TAIL_FULL
    TV = (V + 127) // 128
    NB = 2

    mesh = plsc.VectorSubcoreMesh(core_axis_name="c", subcore_axis_name="s")

    @functools.partial(
        pl.kernel,
        mesh=mesh,
        out_type=jax.ShapeDtypeStruct((ET * TV * 8, 128), jnp.float32),
        compiler_params=pltpu.CompilerParams(use_tc_tiling_on_sc=True),
        scratch_types=[pltpu.VMEM((NB, 8, W), jnp.float32)]
        + [pltpu.SemaphoreType.DMA] * (2 * NB),
    )
    def k1a(tt_hbm, inter_hbm, stage_v, *sems):
        g_sems, w_sems = sems[:NB], sems[NB:]
        wid = lax.axis_index("s") * NC + lax.axis_index("c")
        n_my = (NTASK - 1 - wid) // NW + 1

        def read_dma(j, b):
            bi = wid + j * NW
            et = bi % ET
            v0 = (bi // ET) * W
            v0 = pl.multiple_of(v0, W)
            return pltpu.async_copy(
                tt_hbm.at[pl.ds(et * 8, 8), pl.ds(v0, W)],
                stage_v.at[b],
                g_sems[b],
            )

        def write_tiles(j, u):
            bi = wid + j * NW
            et = bi % ET
            vt0 = (bi // ET) * (W // 128)
            for c in range(W // 128):
                line = (et * TV + vt0 + c) * 8
                line = pl.multiple_of(line, 8)
                _ = pltpu.async_copy(
                    stage_v.at[u, pl.ds(0, 8), pl.ds(c * 128, 128)],
                    inter_hbm.at[pl.ds(line, 8)],
                    w_sems[u],
                )

        def drain_writes(u):
            for c in range(W // 128):
                pltpu.make_async_copy(
                    stage_v.at[u, pl.ds(0, 8), pl.ds(0, 128)],
                    inter_hbm.at[pl.ds(0, 8)],
                    w_sems[u],
                ).wait()

        def body(j2, carry):
            for u in range(NB):
                j = j2 * NB + u

                @pl.when(j + 1 < n_my)
                def _(j=j, b=(u + 1) % NB):
                    @pl.when(j + 1 >= NB)
                    def _():
                        drain_writes(b)

                    _ = read_dma(j + 1, b)

                @pl.when(j < n_my)
                def _(j=j, u=u):
                    pltpu.make_async_copy(
                        tt_hbm.at[pl.ds(0, 8), pl.ds(0, W)],
                        stage_v.at[u],
                        g_sems[u],
                    ).wait()
                    write_tiles(j, u)
            return carry

        @pl.when(n_my > 0)
        def _():
            _ = read_dma(0, 0)

        lax.fori_loop(0, (n_my + NB - 1) // NB, body, 0)

        for u in range(NB):
            @pl.when(n_my >= u + 1)
            def _(u=u):
                drain_writes(u)


        if TAIL_FULL:
            @pl.when(wid < ET)
            def _():
                et = wid
                if TAIL_FULL:
                    nt = TAIL_FULL // 128
                    pltpu.sync_copy(
                        tt_hbm.at[pl.ds(et * 8, 8), pl.ds(V_TAIL0, TAIL_FULL)],
                        stage_v.at[0, pl.ds(0, 8), pl.ds(0, TAIL_FULL)],
                    )
                    for c in range(nt):
                        pltpu.sync_copy(
                            stage_v.at[0, pl.ds(0, 8), pl.ds(c * 128, 128)],
                            inter_hbm.at[
                                pl.ds((et * TV + V_TAIL0 // 128 + c) * 8, 8)
                            ],
                        )

    return k1a


def _scale_table_call(V, NC, NS):
    NW = NC * NS
    VT = V // 128
    V_REM = V - VT * 128
    TV = (V + 127) // 128
    ET = EMBED // 8
    NB = 4
    LA = 2

    mesh = plsc.VectorSubcoreMesh(core_axis_name="c", subcore_axis_name="s")

    @functools.partial(
        pl.kernel,
        mesh=mesh,
        out_type=jax.ShapeDtypeStruct((V * EMBED // 128, 128), jnp.float32),
        compiler_params=pltpu.CompilerParams(
            use_tc_tiling_on_sc=False, needs_layout_passes=False
        ),
        scratch_types=[
            pltpu.VMEM((NB, EMBED, 133), jnp.float32),
            pltpu.VMEM((NB, EMBED, 128), jnp.float32),
            pltpu.VMEM((128, EMBED), jnp.float32),
        ]
        + [pltpu.SemaphoreType.DMA] * (2 * NB),
    )
    def k1b(inter_hbm, tail_hbm, scratch_hbm, stage_v, o_v, tail_v, *sems):
        g_sems, w_sems = sems[:NB], sems[NB:]
        wid = lax.axis_index("s") * NC + lax.axis_index("c")
        n_my = (VT - 1 - wid) // NW + 1
        iota = lax.iota(jnp.int32, LANES)
        e_lo = iota
        e_hi = iota + LANES

        def read_dma(j, b):
            vt = wid + j * NW
            for et in range(ET):
                line = (et * TV + vt) * 8
                line = pl.multiple_of(line, 8)
                _ = pltpu.async_copy(
                    inter_hbm.at[pl.ds(line, 8)],
                    stage_v.at[b, pl.ds(et * 8, 8), pl.ds(0, 128)],
                    g_sems[b],
                )

        def wait_read(b):
            for et in range(ET):
                pltpu.make_async_copy(
                    inter_hbm.at[pl.ds(0, 8)],
                    stage_v.at[b, pl.ds(0, 8), pl.ds(0, 128)],
                    g_sems[b],
                ).wait()

        def transform(b, n_vl):
            bvec = jnp.full((LANES,), b, jnp.int32)
            for f0 in range(0, n_vl * EMBED, LANES):
                vl = f0 // EMBED
                e_vec = e_lo if (f0 % EMBED) == 0 else e_hi
                vals = plsc.load_gather(
                    stage_v, [bvec, e_vec, jnp.full((LANES,), vl, jnp.int32)]
                )
                o_v[b, f0 // 128, pl.ds(f0 % 128, LANES)] = vals * SCALE

        def body(j2, carry):
            for u in range(NB):
                j = j2 * NB + u

                @pl.when(j + LA < n_my)
                def _(j=j, b=(u + LA) % NB):
                    read_dma(j + LA, b)

                @pl.when(jnp.logical_and(j >= NB, j < n_my))
                def _(u=u):
                    pltpu.make_async_copy(
                        o_v.at[u], scratch_hbm.at[pl.ds(0, EMBED)], w_sems[u]
                    ).wait()

                @pl.when(j < n_my)
                def _(j=j, u=u):
                    wait_read(u)
                    transform(u, 128)
                    line0 = (wid + j * NW) * EMBED
                    line0 = pl.multiple_of(line0, EMBED)
                    _ = pltpu.async_copy(
                        o_v.at[u], scratch_hbm.at[pl.ds(line0, EMBED)],
                        w_sems[u],
                    )
            return carry

        for j in range(LA):
            @pl.when(j < n_my)
            def _(j=j):
                read_dma(j, j % NB)

        lax.fori_loop(0, (n_my + NB - 1) // NB, body, 0)

        for u in range(NB):
            @pl.when(n_my >= u + 1)
            def _(u=u):
                pltpu.make_async_copy(
                    o_v.at[u], scratch_hbm.at[pl.ds(0, EMBED)], w_sems[u]
                ).wait()

        if V_REM:
            @pl.when(wid == 0)
            def _():
                pltpu.sync_copy(tail_hbm, tail_v.at[pl.ds(0, V_REM)])
                for r in range(V_REM):
                    for c0 in range(0, EMBED, LANES):
                        f = r * EMBED + c0
                        o_v[0, f // 128, pl.ds(f % 128, LANES)] = (
                            tail_v[r, pl.ds(c0, LANES)] * SCALE
                        )
                pltpu.sync_copy(
                    o_v.at[0, pl.ds(0, V_REM * EMBED // 128), pl.ds(0, 128)],
                    scratch_hbm.at[pl.ds(VT * EMBED, V_REM * EMBED // 128)],
                )

    return k1b


def _gather_call(S, T, V, NC, NS):
    NW = NC * NS
    SB = S // 128
    assert SB == NW
    ET = EMBED // 8
    NB = 4
    LA = 3

    mesh = plsc.VectorSubcoreMesh(core_axis_name="c", subcore_axis_name="s")

    TP = T + 1
    scratch_types_list = [
            pltpu.VMEM((128, TP), jnp.int32),
            pltpu.VMEM((T, 128), jnp.int32),
            pltpu.VMEM((NB, 128, EMBED), jnp.float32),
            pltpu.VMEM((128, EMBED + 1), jnp.float32),
            pltpu.VMEM((NB, ET, 8, 128), jnp.float32),
    ]

    @functools.partial(
        pl.kernel,
        mesh=mesh,
        out_type=jax.ShapeDtypeStruct((T, ET, SB, 8, 128), jnp.float32),
        compiler_params=pltpu.CompilerParams(
            use_tc_tiling_on_sc=False, needs_layout_passes=False
        ),
        scratch_types=scratch_types_list
        + [pltpu.SemaphoreType.DMA] * (2 * NB),
    )
    def k2(tok_hbm, table_hbm, out_hbm, stage_v, tpose_v, rows_v, rp_v, b_v,
           *sems):
        g_sems, w_sems = sems[:NB], sems[NB:]
        wid = lax.axis_index("s") * NC + lax.axis_index("c")
        iota = lax.iota(jnp.int32, LANES)

        pltpu.sync_copy(
            tok_hbm.at[pl.ds(wid * 128, 128)],
            stage_v.at[pl.ds(0, 128), pl.ds(0, T)],
        )
        s_vecs = [iota + c0 for c0 in range(0, 128, LANES)]

        def tpose_body(t, carry):
            tvec = jnp.full((LANES,), t, jnp.int32)
            for ci, sv in enumerate(s_vecs):
                vals = plsc.load_gather(stage_v, [sv, tvec])
                tpose_v[t, pl.ds(ci * LANES, LANES)] = vals
            return carry

        lax.fori_loop(0, T, tpose_body, 0, unroll=2)

        def gather(t, br):
            return pltpu.async_copy(
                table_hbm.at[tpose_v.at[t]], rows_v.at[br], g_sems[br]
            )

        def transform(br):
            for si in range(0, 128):
                for c0 in range(0, EMBED, LANES):
                    rp_v[si, pl.ds(c0, LANES)] = rows_v[
                        br, si, pl.ds(c0, LANES)
                    ]
            for et in range(ET):
                for ei in range(8):
                    evec = jnp.full((LANES,), et * 8 + ei, jnp.int32)
                    for ci, sv in enumerate(s_vecs):
                        vals = plsc.load_gather(rp_v, [sv, evec])
                        b_v[br, et, ei, pl.ds(ci * LANES, LANES)] = vals

        def body(t2, carry):
            for u in range(NB):
                t = t2 * NB + u

                @pl.when(t + LA < T)
                def _(t=t, br=(u + LA) % NB):
                    _ = gather(t + LA, br)

                @pl.when(t >= NB)
                def _(u=u):
                    for et in range(ET):
                        pltpu.make_async_copy(
                            b_v.at[u, et], out_hbm.at[0, et, wid], w_sems[u]
                        ).wait()

                pltpu.make_async_copy(
                    table_hbm.at[tpose_v.at[0]], rows_v.at[u], g_sems[u]
                ).wait()
                transform(u)
                for et in range(ET):
                    _ = pltpu.async_copy(
                        b_v.at[u, et], out_hbm.at[t, et, wid], w_sems[u]
                    )
            return carry

        for t in range(LA):
            _ = gather(t, t)
        lax.fori_loop(0, T // NB, body, 0)
        for u in range(NB):
            for et in range(ET):
                pltpu.make_async_copy(
                    b_v.at[u, et], out_hbm.at[0, et, wid], w_sems[u]
                ).wait()

    return k2


def kernel(tokens, table):
    S, T = tokens.shape
    V = table.shape[0]
    if tokens.dtype != jnp.int32:
        tokens = tokens.astype(jnp.int32)
    info = plsc.get_sparse_core_info()
    NC, NS = info.num_cores, info.num_subcores

    inter = _detile_call(V, NC, NS)(table.T)
    v_full = (V // 128) * 128
    tail = table[v_full:]
    scratch = _scale_table_call(V, NC, NS)(inter, tail)
    scaled = scratch.reshape(V, EMBED)
    out5 = _gather_call(S, T, V, NC, NS)(tokens, scaled)
    return out5.transpose(2, 4, 0, 1, 3).reshape(S, T, EMBED)


# speedup vs baseline: 2.1236x; 1.5777x over previous
import functools
import math

import jax
import jax.numpy as jnp
from jax import lax
from jax.experimental import pallas as pl
from jax.experimental.pallas import tpu as pltpu
from jax.experimental.pallas import tpu_sc as plsc

EMBED = 32
SCALE = math.sqrt(float(EMBED))
LANES = 16


def _detile_call(V, NC, NS):
    NW = NC * NS
    ET = EMBED // 8
    W = 4096
    VB = V // W
    NTASK = ET * VB
    V_TAIL0 = VB * W
    TAIL_FULL = ((V - V_TAIL0) // 128) * 128
    TAIL_REM = V - V_TAIL0 - TAIL_FULL
    TV = (V + 127) // 128
    NB = 2

    mesh = plsc.VectorSubcoreMesh(core_axis_name="c", subcore_axis_name="s")

    @functools.partial(
        pl.kernel,
        mesh=mesh,
        out_type=jax.ShapeDtypeStruct((ET * TV * 8, 128), jnp.float32),
        compiler_params=pltpu.CompilerParams(use_tc_tiling_on_sc=True),
        scratch_types=[pltpu.VMEM((NB, 8, W), jnp.float32)]
        + [pltpu.SemaphoreType.DMA] * (2 * NB),
    )
    def k1a(tt_hbm, inter_hbm, stage_v, *sems):
        g_sems, w_sems = sems[:NB], sems[NB:]
        wid = lax.axis_index("s") * NC + lax.axis_index("c")
        n_my = (NTASK - 1 - wid) // NW + 1

        def read_dma(j, b):
            bi = wid + j * NW
            et = bi % ET
            v0 = (bi // ET) * W
            v0 = pl.multiple_of(v0, W)
            return pltpu.async_copy(
                tt_hbm.at[pl.ds(et * 8, 8), pl.ds(v0, W)],
                stage_v.at[b],
                g_sems[b],
            )

        def write_tiles(j, u):
            bi = wid + j * NW
            et = bi % ET
            vt0 = (bi // ET) * (W // 128)
            for c in range(W // 128):
                line = (et * TV + vt0 + c) * 8
                line = pl.multiple_of(line, 8)
                _ = pltpu.async_copy(
                    stage_v.at[u, pl.ds(0, 8), pl.ds(c * 128, 128)],
                    inter_hbm.at[pl.ds(line, 8)],
                    w_sems[u],
                )

        def drain_writes(u):
            for c in range(W // 128):
                pltpu.make_async_copy(
                    stage_v.at[u, pl.ds(0, 8), pl.ds(0, 128)],
                    inter_hbm.at[pl.ds(0, 8)],
                    w_sems[u],
                ).wait()

        def body(j2, carry):
            for u in range(NB):
                j = j2 * NB + u

                @pl.when(j + 1 < n_my)
                def _(j=j, b=(u + 1) % NB):
                    @pl.when(j + 1 >= NB)
                    def _():
                        drain_writes(b)

                    _ = read_dma(j + 1, b)

                @pl.when(j < n_my)
                def _(j=j, u=u):
                    pltpu.make_async_copy(
                        tt_hbm.at[pl.ds(0, 8), pl.ds(0, W)],
                        stage_v.at[u],
                        g_sems[u],
                    ).wait()
                    write_tiles(j, u)
            return carry

        @pl.when(n_my > 0)
        def _():
            _ = read_dma(0, 0)

        lax.fori_loop(0, (n_my + NB - 1) // NB, body, 0)

        for u in range(NB):
            @pl.when(n_my >= u + 1)
            def _(u=u):
                drain_writes(u)


        if TAIL_FULL:
            @pl.when(wid < ET)
            def _():
                et = wid
                if TAIL_FULL:
                    nt = TAIL_FULL // 128
                    pltpu.sync_copy(
                        tt_hbm.at[pl.ds(et * 8, 8), pl.ds(V_TAIL0, TAIL_FULL)],
                        stage_v.at[0, pl.ds(0, 8), pl.ds(0, TAIL_FULL)],
                    )
                    for c in range(nt):
                        pltpu.sync_copy(
                            stage_v.at[0, pl.ds(0, 8), pl.ds(c * 128, 128)],
                            inter_hbm.at[
                                pl.ds((et * TV + V_TAIL0 // 128 + c) * 8, 8)
                            ],
                        )

    return k1a


def _scale_table_call(V, NC, NS):
    NW = NC * NS
    VT = V // 128
    V_REM = V - VT * 128
    TV = (V + 127) // 128
    ET = EMBED // 8
    NB = 4
    LA = 2

    mesh = plsc.VectorSubcoreMesh(core_axis_name="c", subcore_axis_name="s")

    @functools.partial(
        pl.kernel,
        mesh=mesh,
        out_type=jax.ShapeDtypeStruct((V * EMBED // 128, 128), jnp.float32),
        compiler_params=pltpu.CompilerParams(
            use_tc_tiling_on_sc=False, needs_layout_passes=False
        ),
        scratch_types=[
            pltpu.VMEM((NB, EMBED, 133), jnp.float32),
            pltpu.VMEM((NB, EMBED, 128), jnp.float32),
            pltpu.VMEM((128, EMBED), jnp.float32),
        ]
        + [pltpu.SemaphoreType.DMA] * (2 * NB),
    )
    def k1b(inter_hbm, tail_hbm, scratch_hbm, stage_v, o_v, tail_v, *sems):
        g_sems, w_sems = sems[:NB], sems[NB:]
        wid = lax.axis_index("s") * NC + lax.axis_index("c")
        n_my = (VT - 1 - wid) // NW + 1
        iota = lax.iota(jnp.int32, LANES)
        e_lo = iota
        e_hi = iota + LANES

        def read_dma(j, b):
            vt = wid + j * NW
            for et in range(ET):
                line = (et * TV + vt) * 8
                line = pl.multiple_of(line, 8)
                _ = pltpu.async_copy(
                    inter_hbm.at[pl.ds(line, 8)],
                    stage_v.at[b, pl.ds(et * 8, 8), pl.ds(0, 128)],
                    g_sems[b],
                )

        def wait_read(b):
            for et in range(ET):
                pltpu.make_async_copy(
                    inter_hbm.at[pl.ds(0, 8)],
                    stage_v.at[b, pl.ds(0, 8), pl.ds(0, 128)],
                    g_sems[b],
                ).wait()

        def transform(b, n_vl):
            bvec = jnp.full((LANES,), b, jnp.int32)
            G = 8
            for g0 in range(0, n_vl * EMBED, LANES * G):
                vals = []
                for k in range(G):
                    f0 = g0 + k * LANES
                    vl = f0 // EMBED
                    e_vec = e_lo if (f0 % EMBED) == 0 else e_hi
                    vals.append(plsc.load_gather(
                        stage_v,
                        [bvec, e_vec, jnp.full((LANES,), vl, jnp.int32)],
                    ))
                for k in range(G):
                    f0 = g0 + k * LANES
                    o_v[b, f0 // 128, pl.ds(f0 % 128, LANES)] = (
                        vals[k] * SCALE
                    )

        def body(j2, carry):
            for u in range(NB):
                j = j2 * NB + u

                @pl.when(j + LA < n_my)
                def _(j=j, b=(u + LA) % NB):
                    read_dma(j + LA, b)

                @pl.when(jnp.logical_and(j >= NB, j < n_my))
                def _(u=u):
                    pltpu.make_async_copy(
                        o_v.at[u], scratch_hbm.at[pl.ds(0, EMBED)], w_sems[u]
                    ).wait()

                @pl.when(j < n_my)
                def _(j=j, u=u):
                    wait_read(u)
                    transform(u, 128)
                    line0 = (wid + j * NW) * EMBED
                    line0 = pl.multiple_of(line0, EMBED)
                    _ = pltpu.async_copy(
                        o_v.at[u], scratch_hbm.at[pl.ds(line0, EMBED)],
                        w_sems[u],
                    )
            return carry

        for j in range(LA):
            @pl.when(j < n_my)
            def _(j=j):
                read_dma(j, j % NB)

        lax.fori_loop(0, (n_my + NB - 1) // NB, body, 0)

        for u in range(NB):
            @pl.when(n_my >= u + 1)
            def _(u=u):
                pltpu.make_async_copy(
                    o_v.at[u], scratch_hbm.at[pl.ds(0, EMBED)], w_sems[u]
                ).wait()

        if V_REM:
            @pl.when(wid == 0)
            def _():
                pltpu.sync_copy(tail_hbm, tail_v.at[pl.ds(0, V_REM)])
                for r in range(V_REM):
                    for c0 in range(0, EMBED, LANES):
                        f = r * EMBED + c0
                        o_v[0, f // 128, pl.ds(f % 128, LANES)] = (
                            tail_v[r, pl.ds(c0, LANES)] * SCALE
                        )
                pltpu.sync_copy(
                    o_v.at[0, pl.ds(0, V_REM * EMBED // 128), pl.ds(0, 128)],
                    scratch_hbm.at[pl.ds(VT * EMBED, V_REM * EMBED // 128)],
                )

    return k1b


def _gather_call(S, T, V, NC, NS):
    NW = NC * NS
    SB = S // 128
    assert SB == NW
    ET = EMBED // 8
    NB = 4
    LA = 3

    mesh = plsc.VectorSubcoreMesh(core_axis_name="c", subcore_axis_name="s")

    TP = T + 1
    scratch_types_list = [
            pltpu.VMEM((128, TP), jnp.int32),
            pltpu.VMEM((T, 128), jnp.int32),
            pltpu.VMEM((NB, 128, EMBED), jnp.float32),
            pltpu.VMEM((128, EMBED + 1), jnp.float32),
            pltpu.VMEM((NB, ET, 8, 128), jnp.float32),
    ]

    @functools.partial(
        pl.kernel,
        mesh=mesh,
        out_type=jax.ShapeDtypeStruct((T, ET, SB, 8, 128), jnp.float32),
        compiler_params=pltpu.CompilerParams(
            use_tc_tiling_on_sc=False, needs_layout_passes=False
        ),
        scratch_types=scratch_types_list
        + [pltpu.SemaphoreType.DMA] * (2 * NB),
    )
    def k2(tok_hbm, table_hbm, out_hbm, stage_v, tpose_v, rows_v, rp_v, b_v,
           *sems):
        g_sems, w_sems = sems[:NB], sems[NB:]
        wid = lax.axis_index("s") * NC + lax.axis_index("c")
        iota = lax.iota(jnp.int32, LANES)

        pltpu.sync_copy(
            tok_hbm.at[pl.ds(wid * 128, 128)],
            stage_v.at[pl.ds(0, 128), pl.ds(0, T)],
        )
        s_vecs = [iota + c0 for c0 in range(0, 128, LANES)]

        def tpose_body(t, carry):
            tvec = jnp.full((LANES,), t, jnp.int32)
            for ci, sv in enumerate(s_vecs):
                vals = plsc.load_gather(stage_v, [sv, tvec])
                tpose_v[t, pl.ds(ci * LANES, LANES)] = vals
            return carry

        lax.fori_loop(0, T, tpose_body, 0, unroll=2)

        def gather(t, br):
            return pltpu.async_copy(
                table_hbm.at[tpose_v.at[t]], rows_v.at[br], g_sems[br]
            )

        def transform(br):
            for si in range(0, 128):
                for c0 in range(0, EMBED, LANES):
                    rp_v[si, pl.ds(c0, LANES)] = rows_v[
                        br, si, pl.ds(c0, LANES)
                    ]
            for et in range(ET):
                for ei in range(8):
                    evec = jnp.full((LANES,), et * 8 + ei, jnp.int32)
                    vals = [
                        plsc.load_gather(rp_v, [sv, evec]) for sv in s_vecs
                    ]
                    for ci in range(len(s_vecs)):
                        b_v[br, et, ei, pl.ds(ci * LANES, LANES)] = vals[ci]

        def body(t2, carry):
            for u in range(NB):
                t = t2 * NB + u

                @pl.when(t + LA < T)
                def _(t=t, br=(u + LA) % NB):
                    _ = gather(t + LA, br)

                @pl.when(t >= NB)
                def _(u=u):
                    for et in range(ET):
                        pltpu.make_async_copy(
                            b_v.at[u, et], out_hbm.at[0, et, wid], w_sems[u]
                        ).wait()

                pltpu.make_async_copy(
                    table_hbm.at[tpose_v.at[0]], rows_v.at[u], g_sems[u]
                ).wait()
                transform(u)
                for et in range(ET):
                    _ = pltpu.async_copy(
                        b_v.at[u, et], out_hbm.at[t, et, wid], w_sems[u]
                    )
            return carry

        for t in range(LA):
            _ = gather(t, t)
        lax.fori_loop(0, T // NB, body, 0)
        for u in range(NB):
            for et in range(ET):
                pltpu.make_async_copy(
                    b_v.at[u, et], out_hbm.at[0, et, wid], w_sems[u]
                ).wait()

    return k2


def kernel(tokens, table):
    S, T = tokens.shape
    V = table.shape[0]
    if tokens.dtype != jnp.int32:
        tokens = tokens.astype(jnp.int32)
    info = plsc.get_sparse_core_info()
    NC, NS = info.num_cores, info.num_subcores

    inter = _detile_call(V, NC, NS)(table.T)
    v_full = (V // 128) * 128
    tail = table[v_full:]
    scratch = _scale_table_call(V, NC, NS)(inter, tail)
    scaled = scratch.reshape(V, EMBED)
    out5 = _gather_call(S, T, V, NC, NS)(tokens, scaled)
    return out5.transpose(2, 4, 0, 1, 3).reshape(S, T, EMBED)
